# baseline (speedup 1.0000x reference)
import functools

import jax
import jax.numpy as jnp
from jax import lax
from jax.experimental import pallas as pl
from jax.experimental.pallas import tpu as pltpu
from jax.experimental.pallas import tpu_sc as plsc

VOCAB = 1000000
D_MODEL = 64
BATCH = 4096
CTX = 200

NC, NS = 2, 16
NW = NC * NS
TOK = BATCH * CTX
TPW = TOK // NW
G = 128
NG = TPW // G
NBUF = 4

_mesh = plsc.VectorSubcoreMesh(core_axis_name="c", subcore_axis_name="s")


@functools.partial(
    pl.kernel,
    mesh=_mesh,
    compiler_params=pltpu.CompilerParams(use_tc_tiling_on_sc=False),
    out_type=jax.ShapeDtypeStruct((TOK, D_MODEL), jnp.float32),
    scratch_types=[
        pltpu.VMEM((NG, G), jnp.int32),
        pltpu.VMEM((NBUF, G, D_MODEL), jnp.float32),
        pltpu.SemaphoreType.DMA,
        pltpu.SemaphoreType.DMA,
        pltpu.SemaphoreType.DMA,
        pltpu.SemaphoreType.DMA,
        pltpu.SemaphoreType.DMA,
        pltpu.SemaphoreType.DMA,
        pltpu.SemaphoreType.DMA,
        pltpu.SemaphoreType.DMA,
    ],
)
def _emb_sc(x2_hbm, tab_hbm, out_hbm, idx_v, rows_v,
            g0, g1, g2, g3, o0, o1, o2, o3):
    gs = (g0, g1, g2, g3)
    os_ = (o0, o1, o2, o3)
    wid = lax.axis_index("s") * NC + lax.axis_index("c")
    row0 = wid * NG
    tok0 = wid * TPW

    pltpu.sync_copy(x2_hbm.at[pl.ds(row0, NG)], idx_v)

    def gather(i, s):
        pltpu.async_copy(tab_hbm.at[idx_v.at[i]], rows_v.at[s], gs[s])

    def wait_gather(i, s):
        pltpu.make_async_copy(tab_hbm.at[idx_v.at[i]], rows_v.at[s],
                              gs[s]).wait()

    def out_start(i, s):
        pltpu.async_copy(rows_v.at[s],
                         out_hbm.at[pl.ds(tok0 + i * G, G)], os_[s])

    def out_wait(i, s):
        pltpu.make_async_copy(rows_v.at[s],
                              out_hbm.at[pl.ds(tok0 + i * G, G)],
                              os_[s]).wait()

    for j in range(NBUF - 1):
        gather(j, j)

    def round_(r, carry):
        for s in range(NBUF):
            i = r * NBUF + s
            wait_gather(i, s)
            out_start(i, s)
            sj = (s + NBUF - 1) % NBUF
            if s == 0:
                @pl.when(r > 0)
                def _():
                    out_wait(i - 1, sj)
                gather(i + NBUF - 1, sj)
            else:
                @pl.when(r < NG // NBUF - 1)
                def _():
                    out_wait(i - 1, sj)
                    gather(i + NBUF - 1, sj)
        return carry

    lax.fori_loop(0, NG // NBUF, round_, 0)

    for k in range(NBUF):
        out_wait(NG - NBUF + k, k)


def kernel(x, table):
    x2 = x.reshape(NW * NG, G)
    out = _emb_sc(x2, table)
    return out.reshape(BATCH, CTX, D_MODEL)


# speedup vs baseline: 1.0024x; 1.0024x over previous
import functools

import jax
import jax.numpy as jnp
from jax import lax
from jax.experimental import pallas as pl
from jax.experimental.pallas import tpu as pltpu
from jax.experimental.pallas import tpu_sc as plsc

VOCAB = 1000000
D_MODEL = 64
BATCH = 4096
CTX = 200

NC, NS = 2, 16
NW = NC * NS
TOK = BATCH * CTX
TPW = TOK // NW
G = 128
NG = TPW // G
NBUF = 8

_mesh = plsc.VectorSubcoreMesh(core_axis_name="c", subcore_axis_name="s")


@functools.partial(
    pl.kernel,
    mesh=_mesh,
    compiler_params=pltpu.CompilerParams(use_tc_tiling_on_sc=False),
    out_type=jax.ShapeDtypeStruct((TOK, D_MODEL), jnp.float32),
    scratch_types=(
        [pltpu.VMEM((NG, G), jnp.int32),
         pltpu.VMEM((NBUF, G, D_MODEL), jnp.float32)]
        + [pltpu.SemaphoreType.DMA] * (2 * NBUF)
    ),
)
def _emb_sc(x2_hbm, tab_hbm, out_hbm, idx_v, rows_v, *sems):
    gs = sems[:NBUF]
    os_ = sems[NBUF:]
    wid = lax.axis_index("s") * NC + lax.axis_index("c")
    row0 = wid * NG
    tok0 = wid * TPW

    pltpu.sync_copy(x2_hbm.at[pl.ds(row0, NG)], idx_v)

    def gather(i, s):
        pltpu.async_copy(tab_hbm.at[idx_v.at[i]], rows_v.at[s], gs[s])

    def wait_gather(i, s):
        pltpu.make_async_copy(tab_hbm.at[idx_v.at[i]], rows_v.at[s],
                              gs[s]).wait()

    def out_start(i, s):
        pltpu.async_copy(rows_v.at[s],
                         out_hbm.at[pl.ds(tok0 + i * G, G)], os_[s])

    def out_wait(i, s):
        pltpu.make_async_copy(rows_v.at[s],
                              out_hbm.at[pl.ds(tok0 + i * G, G)],
                              os_[s]).wait()

    for j in range(NBUF - 1):
        gather(j, j)

    def round_(r, carry):
        for s in range(NBUF):
            i = r * NBUF + s
            wait_gather(i, s)
            out_start(i, s)
            sj = (s + NBUF - 1) % NBUF
            if s == 0:
                @pl.when(r > 0)
                def _():
                    out_wait(i - 1, sj)
                gather(i + NBUF - 1, sj)
            else:
                @pl.when(r < NG // NBUF - 1)
                def _():
                    out_wait(i - 1, sj)
                    gather(i + NBUF - 1, sj)
        return carry

    lax.fori_loop(0, NG // NBUF, round_, 0)

    for k in range(NBUF):
        out_wait(NG - NBUF + k, k)


def kernel(x, table):
    x2 = x.reshape(NW * NG, G)
    out = _emb_sc(x2, table)
    return out.reshape(BATCH, CTX, D_MODEL)
